# Initial kernel scaffold; baseline (speedup 1.0000x reference)
import functools

import jax
import jax.numpy as jnp
from jax import lax
from jax.experimental import pallas as pl
from jax.experimental.pallas import tpu as pltpu
from jax.experimental.pallas import tpu_sc as plsc

NC = 2
NS = 16
NW = NC * NS


@functools.lru_cache(maxsize=None)
def _build_gather(B: int, V: int, D: int):
    assert B % NW == 0
    b_per_w = B // NW
    chunk = 1600
    while b_per_w % chunk:
        chunk //= 2
    n_chunks = b_per_w // chunk

    mesh = plsc.VectorSubcoreMesh(core_axis_name="c", subcore_axis_name="s")

    @functools.partial(
        pl.kernel,
        out_type=jax.ShapeDtypeStruct((B, D), jnp.float32),
        mesh=mesh,
        scratch_types=[
            pltpu.VMEM((chunk,), jnp.int32),
            pltpu.VMEM((chunk, D), jnp.float32),
            pltpu.SemaphoreType.DMA,
        ],
    )
    def gather(idx_hbm, table_hbm, out_hbm, idx_v, rows_v, sem):
        wid = lax.axis_index("s") * NC + lax.axis_index("c")
        base = wid * b_per_w

        @pl.loop(0, n_chunks)
        def _(i):
            off = base + i * chunk
            pltpu.sync_copy(idx_hbm.at[pl.ds(off, chunk)], idx_v)
            pltpu.async_copy(table_hbm.at[idx_v], rows_v, sem).wait()
            pltpu.sync_copy(rows_v, out_hbm.at[pl.ds(off, chunk)])

    return gather


def kernel(x, w):
    V, D = w.shape
    x_flat = x.reshape(-1)
    out = _build_gather(x_flat.shape[0], V, D)(x_flat, w)
    return out.reshape(x.shape + (D,))


# speedup vs baseline: 1.4766x; 1.4766x over previous
import functools

import jax
import jax.numpy as jnp
from jax import lax
from jax.experimental import pallas as pl
from jax.experimental.pallas import tpu as pltpu
from jax.experimental.pallas import tpu_sc as plsc

NC = 2
NS = 16
NW = NC * NS


@functools.lru_cache(maxsize=None)
def _build_gather(B: int, V: int, D: int):
    assert B % NW == 0
    b_per_w = B // NW
    chunk = 1600
    while b_per_w % chunk:
        chunk //= 2
    n_chunks = b_per_w // chunk

    mesh = plsc.VectorSubcoreMesh(core_axis_name="c", subcore_axis_name="s")

    @functools.partial(
        pl.kernel,
        out_type=jax.ShapeDtypeStruct((B, D), jnp.float32),
        mesh=mesh,
        scratch_types=[
            pltpu.VMEM((chunk,), jnp.int32),
            pltpu.VMEM((chunk, D), jnp.float32),
            pltpu.SemaphoreType.DMA,
        ],
        compiler_params=pltpu.CompilerParams(use_tc_tiling_on_sc=False),
    )
    def gather(idx_hbm, table_hbm, out_hbm, idx_v, rows_v, sem):
        wid = lax.axis_index("s") * NC + lax.axis_index("c")
        base = wid * b_per_w

        @pl.loop(0, n_chunks)
        def _(i):
            off = base + i * chunk
            pltpu.sync_copy(idx_hbm.at[pl.ds(off, chunk)], idx_v)
            pltpu.async_copy(table_hbm.at[idx_v], rows_v, sem).wait()
            pltpu.sync_copy(rows_v, out_hbm.at[pl.ds(off, chunk)])

    return gather


def kernel(x, w):
    V, D = w.shape
    x_flat = x.reshape(-1)
    out = _build_gather(x_flat.shape[0], V, D)(x_flat, w)
    return out.reshape(x.shape + (D,))


# speedup vs baseline: 1.5004x; 1.0161x over previous
import functools

import jax
import jax.numpy as jnp
from jax import lax
from jax.experimental import pallas as pl
from jax.experimental.pallas import tpu as pltpu
from jax.experimental.pallas import tpu_sc as plsc

NC = 2
NS = 16
NW = NC * NS


@functools.lru_cache(maxsize=None)
def _build_gather(B: int, V: int, D: int):
    assert B % NW == 0
    b_per_w = B // NW
    chunk = 1600
    while b_per_w % chunk:
        chunk //= 2
    n_chunks = b_per_w // chunk

    mesh = plsc.VectorSubcoreMesh(core_axis_name="c", subcore_axis_name="s")

    @functools.partial(
        pl.kernel,
        out_type=jax.ShapeDtypeStruct((B, D), jnp.float32),
        mesh=mesh,
        scratch_types=[
            pltpu.VMEM((b_per_w,), jnp.int32),
            pltpu.VMEM((chunk, D), jnp.float32),
            pltpu.VMEM((chunk, D), jnp.float32),
            pltpu.SemaphoreType.DMA,
            pltpu.SemaphoreType.DMA,
            pltpu.SemaphoreType.DMA,
            pltpu.SemaphoreType.DMA,
        ],
        compiler_params=pltpu.CompilerParams(use_tc_tiling_on_sc=False),
    )
    def gather(idx_hbm, table_hbm, out_hbm, idx_all, rows0, rows1,
               sg0, sg1, ss0, ss1):
        wid = lax.axis_index("s") * NC + lax.axis_index("c")
        base = wid * b_per_w
        rows = (rows0, rows1)
        sg = (sg0, sg1)
        ss = (ss0, ss1)

        pltpu.sync_copy(idx_hbm.at[pl.ds(base, b_per_w)], idx_all)

        def idx_slice(j):
            return idx_all.at[pl.ds(j * chunk, chunk)]

        def start_gather(j, b):
            pltpu.async_copy(table_hbm.at[idx_slice(j)], rows[b], sg[b])

        def wait_gather(b):
            pltpu.make_async_copy(table_hbm.at[idx_slice(0)], rows[b],
                                  sg[b]).wait()

        def start_store(j, b):
            pltpu.async_copy(rows[b], out_hbm.at[pl.ds(base + j * chunk, chunk)],
                             ss[b])

        def wait_store(b):
            pltpu.make_async_copy(rows[b], out_hbm.at[pl.ds(base, chunk)],
                                  ss[b]).wait()

        start_gather(0, 0)
        start_gather(1, 1)

        @pl.loop(0, n_chunks - 2, step=2)
        def _(i):
            for b in range(2):
                j = i + b
                wait_gather(b)
                start_store(j, b)
                wait_store(b)
                start_gather(j + 2, b)

        for b in range(2):
            j = n_chunks - 2 + b
            wait_gather(b)
            start_store(j, b)
        for b in range(2):
            wait_store(b)

    return gather


def kernel(x, w):
    V, D = w.shape
    x_flat = x.reshape(-1)
    out = _build_gather(x_flat.shape[0], V, D)(x_flat, w)
    return out.reshape(x.shape + (D,))


# speedup vs baseline: 2.0509x; 1.3668x over previous
import functools

import jax
import jax.numpy as jnp
from jax import lax
from jax.experimental import pallas as pl
from jax.experimental.pallas import tpu as pltpu
from jax.experimental.pallas import tpu_sc as plsc

NC = 2
NS = 16
NW = NC * NS


@functools.lru_cache(maxsize=None)
def _build_gather(B: int, V: int, D: int):
    assert B % NW == 0
    b_per_w = B // NW
    chunk = 1600
    while b_per_w % chunk:
        chunk //= 2
    n_chunks = b_per_w // chunk

    mesh = plsc.VectorSubcoreMesh(core_axis_name="c", subcore_axis_name="s")

    @functools.partial(
        pl.kernel,
        out_type=jax.ShapeDtypeStruct((B, 128), jnp.float32),
        mesh=mesh,
        scratch_types=[
            pltpu.VMEM((b_per_w,), jnp.int32),
            pltpu.VMEM((chunk, D), jnp.float32),
            pltpu.VMEM((chunk, D), jnp.float32),
            pltpu.SemaphoreType.DMA,
            pltpu.SemaphoreType.DMA,
            pltpu.SemaphoreType.DMA,
            pltpu.SemaphoreType.DMA,
        ],
        compiler_params=pltpu.CompilerParams(use_tc_tiling_on_sc=False),
    )
    def gather(idx_hbm, table_hbm, out_hbm, idx_all, rows0, rows1,
               sg0, sg1, ss0, ss1):
        wid = lax.axis_index("s") * NC + lax.axis_index("c")
        base = wid * b_per_w
        rows = (rows0, rows1)
        sg = (sg0, sg1)
        ss = (ss0, ss1)

        pltpu.sync_copy(idx_hbm.at[pl.ds(base, b_per_w)], idx_all)

        def idx_slice(j):
            return idx_all.at[pl.ds(j * chunk, chunk)]

        def start_gather(j, b):
            pltpu.async_copy(table_hbm.at[idx_slice(j)], rows[b], sg[b])

        def wait_gather(b):
            pltpu.make_async_copy(table_hbm.at[idx_slice(0)], rows[b],
                                  sg[b]).wait()

        def start_store(j, b):
            pltpu.async_copy(
                rows[b],
                out_hbm.at[pl.ds(base + j * chunk, chunk), pl.ds(0, D)],
                ss[b])

        def wait_store(b):
            pltpu.make_async_copy(
                rows[b], out_hbm.at[pl.ds(base, chunk), pl.ds(0, D)],
                ss[b]).wait()

        start_gather(0, 0)
        start_gather(1, 1)

        @pl.loop(0, n_chunks - 2, step=2)
        def _(i):
            for b in range(2):
                j = i + b
                wait_gather(b)
                start_store(j, b)
                wait_store(b)
                start_gather(j + 2, b)

        for b in range(2):
            j = n_chunks - 2 + b
            wait_gather(b)
            start_store(j, b)
        for b in range(2):
            wait_store(b)

    return gather


def kernel(x, w):
    V, D = w.shape
    x_flat = x.reshape(-1)
    out = _build_gather(x_flat.shape[0], V, D)(x_flat, w)
    return out[:, :D].reshape(x.shape + (D,))
